# baseline (speedup 1.0000x reference)
import functools
import math

import jax
import jax.numpy as jnp
from jax import lax
from jax.experimental import pallas as pl
from jax.experimental.pallas import tpu as pltpu
from jax.experimental.pallas import tpu_sc as plsc

_R2 = 3200


def _pe_table2(L, D):
    D2 = 2 * D

    def body(o_ref):
        p = lax.broadcasted_iota(jnp.int32, (L, D2), 0).astype(jnp.float32)
        j = lax.broadcasted_iota(jnp.int32, (L, D2), 1)
        d = j % D
        half = (d // 2).astype(jnp.float32)
        ang = p * jnp.exp(half * (-2.0 * math.log(10000.0) / D))
        o_ref[...] = jnp.where(d % 2 == 0, jnp.sin(ang), jnp.cos(ang))

    return pl.pallas_call(
        body, out_shape=jax.ShapeDtypeStruct((L, D2), jnp.float32)
    )()


def _sc_gather2(idxE, idxO, tbl_lin):
    NW, n_ch, CH = idxE.shape
    V, D = tbl_lin.shape
    D2 = 2 * D
    NBUF = 5
    per_w = n_ch * CH
    info = plsc.get_sparse_core_info()
    NC = info.num_cores
    mesh = plsc.VectorSubcoreMesh(core_axis_name="c", subcore_axis_name="s")

    @functools.partial(
        pl.kernel,
        out_type=jax.ShapeDtypeStruct((NW * per_w, D2), jnp.float32),
        mesh=mesh,
        compiler_params=pltpu.CompilerParams(use_tc_tiling_on_sc=False),
        scratch_types=(
            [pltpu.VMEM((n_ch, CH), jnp.int32)] * 2
            + [pltpu.VMEM((CH, D), jnp.float32)] * (2 * NBUF)
            + [pltpu.SemaphoreType.DMA] * (2 * NBUF)
        ),
    )
    def k(idxE_hbm, idxO_hbm, tbl_hbm, out_hbm, idxE_v, idxO_v, *bufs_sems):
        ebufs = list(bufs_sems[0:NBUF])
        obufs = list(bufs_sems[NBUF:2 * NBUF])
        gs = list(bufs_sems[2 * NBUF:3 * NBUF])
        ss = list(bufs_sems[3 * NBUF:4 * NBUF])
        tbl = tbl_hbm
        wid = lax.axis_index("s") * NC + lax.axis_index("c")
        base = wid * per_w
        pltpu.sync_copy(idxE_hbm.at[wid], idxE_v)
        pltpu.sync_copy(idxO_hbm.at[wid], idxO_v)

        def fire_gather(t, b):
            pltpu.async_copy(tbl.at[idxE_v.at[t]], ebufs[b], gs[b])
            pltpu.async_copy(tbl.at[idxO_v.at[t]], obufs[b], gs[b])

        def wait_gather(b):
            pltpu.make_async_copy(tbl.at[idxE_v.at[0]], ebufs[b], gs[b]).wait()
            pltpu.make_async_copy(tbl.at[idxO_v.at[0]], obufs[b], gs[b]).wait()

        def fire_store(t, b):
            r0 = base + t * CH
            pltpu.async_copy(
                ebufs[b], out_hbm.at[pl.ds(r0, CH), pl.ds(0, D)], ss[b])
            pltpu.async_copy(
                obufs[b], out_hbm.at[pl.ds(r0, CH), pl.ds(D, D)], ss[b])

        def wait_store(b):
            pltpu.make_async_copy(
                ebufs[b], out_hbm.at[pl.ds(base, CH), pl.ds(0, D)], ss[b]).wait()
            pltpu.make_async_copy(
                obufs[b], out_hbm.at[pl.ds(base, CH), pl.ds(D, D)], ss[b]).wait()

        fire_gather(0, 0)
        fire_gather(1, 1)

        def outer(ti, _):
            for h in range(NBUF):
                t = ti * NBUF + h
                wait_gather(h)
                fire_store(t, h)
                b2 = (h + 2) % NBUF

                @pl.when(t + 2 >= NBUF)
                def _():
                    wait_store(b2)

                @pl.when(t + 2 < n_ch)
                def _():
                    fire_gather(t + 2, b2)
            return ()

        lax.fori_loop(0, n_ch // NBUF, outer, ())
        for u in range(NBUF - 2):
            wait_store((n_ch - (NBUF - 2) + u) % NBUF)

    return k(idxE, idxO, tbl_lin)


def _ln_tc2(rows2, pe2, g2, b2, M):
    N2, D2 = rows2.shape
    L2 = pe2.shape[0]
    D = D2 // 2
    R2 = _R2
    reps = R2 // L2

    def body(x_ref, pe_ref, g_ref, b_ref, m_ref, o_ref):
        x = x_ref[...]
        e = (x.reshape(reps, L2, D2) + pe_ref[...][None]).reshape(R2, D2)
        M_ = m_ref[...]
        m = jnp.dot(e, M_, preferred_element_type=jnp.float32,
                    precision=lax.Precision.DEFAULT)
        c = e - m
        v = jnp.dot(c * c, M_, preferred_element_type=jnp.float32,
                    precision=lax.Precision.DEFAULT)
        o_ref[...] = c * lax.rsqrt(v + 1e-5) * g_ref[...] + b_ref[...]

    return pl.pallas_call(
        body,
        grid=(N2 // R2,),
        in_specs=[
            pl.BlockSpec((R2, D2), lambda i: (i, 0)),
            pl.BlockSpec((L2, D2), lambda i: (0, 0)),
            pl.BlockSpec((1, D2), lambda i: (0, 0)),
            pl.BlockSpec((1, D2), lambda i: (0, 0)),
            pl.BlockSpec((D2, D2), lambda i: (0, 0)),
        ],
        out_specs=pl.BlockSpec((R2, D2), lambda i: (i, 0)),
        out_shape=jax.ShapeDtypeStruct((N2, D2), jnp.float32),
    )(rows2, pe2, g2, b2, M)


def kernel(input_dp, table, ln_gamma, ln_beta):
    B, L = input_dp.shape
    V, D = table.shape
    NW, CH = 32, 128
    K = 4
    flat = input_dp.reshape(-1).astype(jnp.int32)
    N2 = flat.shape[0] // 2
    NS = N2 // K
    n_ch = NS // (NW * CH)
    grp = flat.reshape(N2 // _R2, 2, _R2)
    idxE = grp[:, 0, :].reshape(K, NW, n_ch, CH)
    idxO = grp[:, 1, :].reshape(K, NW, n_ch, CH)
    pe2 = _pe_table2(L, D)
    D2 = 2 * D
    g2 = jnp.concatenate([ln_gamma, ln_gamma]).reshape(1, D2)
    b2 = jnp.concatenate([ln_beta, ln_beta]).reshape(1, D2)
    ii = jnp.arange(D2)[:, None]
    jj = jnp.arange(D2)[None, :]
    M = jnp.where((ii // D) == (jj // D), 1.0 / D, 0.0).astype(jnp.float32)
    normed = [
        _ln_tc2(_sc_gather2(idxE[k], idxO[k], table), pe2, g2, b2, M)
        for k in range(K)
    ]
    n_grp = NS // _R2
    hb = _R2 // L
    parts = []
    for x in normed:
        x6 = x.reshape(n_grp, hb, L, 2, D)
        lo = x6[:, :, :, 0, :]
        hi = x6[:, :, :, 1, :]
        parts.append(
            jnp.concatenate([lo, hi], axis=1).reshape(B // K, L, D))
    return jnp.concatenate(parts, axis=0)


# speedup vs baseline: 2.6115x; 2.6115x over previous
import functools
import math

import jax
import jax.numpy as jnp
from jax import lax
from jax.experimental import pallas as pl
from jax.experimental.pallas import tpu as pltpu
from jax.experimental.pallas import tpu_sc as plsc

_R2 = 3200


def _pe_table2(L, D):
    D2 = 2 * D

    def body(o_ref):
        p = lax.broadcasted_iota(jnp.int32, (L, D2), 0).astype(jnp.float32)
        j = lax.broadcasted_iota(jnp.int32, (L, D2), 1)
        d = j % D
        half = (d // 2).astype(jnp.float32)
        ang = p * jnp.exp(half * (-2.0 * math.log(10000.0) / D))
        o_ref[...] = jnp.where(d % 2 == 0, jnp.sin(ang), jnp.cos(ang))

    return pl.pallas_call(
        body, out_shape=jax.ShapeDtypeStruct((L, D2), jnp.float32)
    )()


def _sc_gather2(idxE, idxO, tbl_lin):
    NW, n_ch, CH = idxE.shape
    V, D = tbl_lin.shape
    D2 = 2 * D
    NBUF = 5
    per_w = n_ch * CH
    info = plsc.get_sparse_core_info()
    NC = info.num_cores
    mesh = plsc.VectorSubcoreMesh(core_axis_name="c", subcore_axis_name="s")

    @functools.partial(
        pl.kernel,
        out_type=jax.ShapeDtypeStruct((NW * per_w, D2), jnp.float32),
        mesh=mesh,
        compiler_params=pltpu.CompilerParams(use_tc_tiling_on_sc=False),
        scratch_types=(
            [pltpu.VMEM((n_ch, CH), jnp.int32)] * 2
            + [pltpu.VMEM((CH, D), jnp.float32)] * (2 * NBUF)
            + [pltpu.SemaphoreType.DMA] * (2 * NBUF)
        ),
    )
    def k(idxE_hbm, idxO_hbm, tbl_hbm, out_hbm, idxE_v, idxO_v, *bufs_sems):
        ebufs = list(bufs_sems[0:NBUF])
        obufs = list(bufs_sems[NBUF:2 * NBUF])
        gs = list(bufs_sems[2 * NBUF:3 * NBUF])
        ss = list(bufs_sems[3 * NBUF:4 * NBUF])
        tbl = tbl_hbm
        wid = lax.axis_index("s") * NC + lax.axis_index("c")
        base = wid * per_w
        pltpu.sync_copy(idxE_hbm.at[wid], idxE_v)
        pltpu.sync_copy(idxO_hbm.at[wid], idxO_v)

        def fire_gather(t, b):
            pltpu.async_copy(tbl.at[idxE_v.at[t]], ebufs[b], gs[b])
            pltpu.async_copy(tbl.at[idxO_v.at[t]], obufs[b], gs[b])

        def wait_gather(b):
            pltpu.make_async_copy(tbl.at[idxE_v.at[0]], ebufs[b], gs[b]).wait()
            pltpu.make_async_copy(tbl.at[idxO_v.at[0]], obufs[b], gs[b]).wait()

        def fire_store(t, b):
            r0 = base + t * CH
            pltpu.async_copy(
                ebufs[b], out_hbm.at[pl.ds(r0, CH), pl.ds(0, D)], ss[b])
            pltpu.async_copy(
                obufs[b], out_hbm.at[pl.ds(r0, CH), pl.ds(D, D)], ss[b])

        def wait_store(b):
            pltpu.make_async_copy(
                ebufs[b], out_hbm.at[pl.ds(base, CH), pl.ds(0, D)], ss[b]).wait()
            pltpu.make_async_copy(
                obufs[b], out_hbm.at[pl.ds(base, CH), pl.ds(D, D)], ss[b]).wait()

        fire_gather(0, 0)
        fire_gather(1, 1)

        def outer(ti, _):
            for h in range(NBUF):
                t = ti * NBUF + h
                wait_gather(h)
                fire_store(t, h)
                b2 = (h + 2) % NBUF

                @pl.when(t + 2 >= NBUF)
                def _():
                    wait_store(b2)

                @pl.when(t + 2 < n_ch)
                def _():
                    fire_gather(t + 2, b2)
            return ()

        lax.fori_loop(0, n_ch // NBUF, outer, ())
        for u in range(NBUF - 2):
            wait_store((n_ch - (NBUF - 2) + u) % NBUF)

    return k(idxE, idxO, tbl_lin)


def _ln_tc2(rows2, pe2, g2, b2, M, B, L, blk_off, outbuf):
    N2, D2 = rows2.shape
    L2 = pe2.shape[0]
    D = D2 // 2
    R2 = _R2
    reps = R2 // L2
    BL = (2 * R2) // L

    def body(*refs):
        x_ref, pe_ref, g_ref, b_ref, m_ref = refs[-6:-1]
        o_ref = refs[-1]
        x = x_ref[...]
        e = (x.reshape(reps, L2, D2) + pe_ref[...][None]).reshape(R2, D2)
        M_ = m_ref[...]
        m = jnp.dot(e, M_, preferred_element_type=jnp.float32,
                    precision=lax.Precision.DEFAULT)
        c = e - m
        v = jnp.dot(c * c, M_, preferred_element_type=jnp.float32,
                    precision=lax.Precision.DEFAULT)
        o = c * lax.rsqrt(v + 1e-5) * g_ref[...] + b_ref[...]
        half = BL // 2
        o_ref[...] = jnp.concatenate(
            [o[:, :D].reshape(half, L, D), o[:, D:].reshape(half, L, D)],
            axis=0,
        )

    in_specs = [
        pl.BlockSpec((R2, D2), lambda i: (i, 0)),
        pl.BlockSpec((L2, D2), lambda i: (0, 0)),
        pl.BlockSpec((1, D2), lambda i: (0, 0)),
        pl.BlockSpec((1, D2), lambda i: (0, 0)),
        pl.BlockSpec((D2, D2), lambda i: (0, 0)),
    ]
    args = (rows2, pe2, g2, b2, M)
    aliases = {}
    if outbuf is not None:
        in_specs = [pl.BlockSpec(memory_space=pl.ANY)] + in_specs
        args = (outbuf,) + args
        aliases = {0: 0}
    return pl.pallas_call(
        body,
        grid=(N2 // R2,),
        in_specs=in_specs,
        out_specs=pl.BlockSpec((BL, L, D), lambda i: (blk_off + i, 0, 0)),
        out_shape=jax.ShapeDtypeStruct((B, L, D), jnp.float32),
        input_output_aliases=aliases,
    )(*args)


def kernel(input_dp, table, ln_gamma, ln_beta):
    B, L = input_dp.shape
    V, D = table.shape
    NW, CH = 32, 128
    K = 4
    flat = input_dp.reshape(-1).astype(jnp.int32)
    N2 = flat.shape[0] // 2
    NS = N2 // K
    n_ch = NS // (NW * CH)
    grp = flat.reshape(N2 // _R2, 2, _R2)
    idxE = grp[:, 0, :].reshape(K, NW, n_ch, CH)
    idxO = grp[:, 1, :].reshape(K, NW, n_ch, CH)
    pe2 = _pe_table2(L, D)
    D2 = 2 * D
    g2 = jnp.concatenate([ln_gamma, ln_gamma]).reshape(1, D2)
    b2 = jnp.concatenate([ln_beta, ln_beta]).reshape(1, D2)
    ii = jnp.arange(D2)[:, None]
    jj = jnp.arange(D2)[None, :]
    M = jnp.where((ii // D) == (jj // D), 1.0 / D, 0.0).astype(jnp.float32)
    rows = [_sc_gather2(idxE[k], idxO[k], table) for k in range(K)]
    blk_per_k = (2 * NS) // (L * ((2 * _R2) // L))
    out = None
    for k in range(K):
        out = _ln_tc2(rows[k], pe2, g2, b2, M, B, L, k * blk_per_k, out)
    return out
